# Initial kernel scaffold; baseline (speedup 1.0000x reference)
import jax
import jax.numpy as jnp
from jax.experimental import pallas as pl


def kernel(x):
    raise NotImplementedError("write your pallas kernel here")


# speedup vs baseline: 287.1681x; 287.1681x over previous
import functools

import jax
import jax.numpy as jnp
from jax import lax
from jax.experimental import pallas as pl
from jax.experimental.pallas import tpu as pltpu
from jax.experimental.pallas import tpu_sc as plsc

_B, _C, _H, _W = 8, 384, 32, 32
_NZ = 2
_HO = _H * (_NZ + 1) - _NZ
_WO = _W * (_NZ + 1) - _NZ
_PAIRS = _B * _C
_IN_PP = _H * _W
_OUT_PP = _HO * _WO

_NW = 32
_PAIRS_PER_W = _PAIRS // _NW
_UNIT_PAIRS = 2
_UNITS = _PAIRS_PER_W // _UNIT_PAIRS
_UNIT_IN = _UNIT_PAIRS * _IN_PP
_UNIT_OUT = _UNIT_PAIRS * _OUT_PP


@functools.partial(
    pl.kernel,
    out_type=jax.ShapeDtypeStruct((_PAIRS * _OUT_PP,), jnp.float32),
    mesh=plsc.VectorSubcoreMesh(core_axis_name="c", subcore_axis_name="s"),
    scratch_types=[
        pltpu.VMEM((_UNIT_IN,), jnp.float32),
        pltpu.VMEM((_UNIT_OUT,), jnp.float32),
    ],
    compiler_params=pltpu.CompilerParams(needs_layout_passes=False),
)
def _weave_sc(x_hbm, out_hbm, xbuf, obuf):
    nc = 2
    wid = lax.axis_index("s") * nc + lax.axis_index("c")
    base_pair = wid * _PAIRS_PER_W

    zeros16 = jnp.zeros((16,), jnp.float32)

    def _zero(i, c):
        obuf[pl.ds(i * 16, 16)] = zeros16
        return c

    lax.fori_loop(0, _UNIT_OUT // 16, _zero, 0)
    obuf[pl.ds(_UNIT_OUT - 16, 16)] = zeros16

    iota = lax.iota(jnp.int32, 16)
    col_lo = iota * 3
    col_hi = col_lo + 48

    def _unit(u, c):
        p0 = base_pair + u * _UNIT_PAIRS
        pltpu.sync_copy(x_hbm.at[pl.ds(p0 * _IN_PP, _UNIT_IN)], xbuf)
        for p in range(_UNIT_PAIRS):
            for r in range(_H):
                src = p * _IN_PP + r * _W
                row_lo = xbuf[pl.ds(src, 16)]
                row_hi = xbuf[pl.ds(src + 16, 16)]
                dst = p * _OUT_PP + (3 * r) * _WO
                plsc.store_scatter(obuf, [col_lo + dst], row_lo)
                plsc.store_scatter(obuf, [col_hi + dst], row_hi)
        pltpu.sync_copy(obuf, out_hbm.at[pl.ds(p0 * _OUT_PP, _UNIT_OUT)])
        return c

    lax.fori_loop(0, _UNITS, _unit, 0)


def kernel(x):
    out_flat = _weave_sc(x.reshape(_PAIRS * _IN_PP))
    return out_flat.reshape(_B, _C, _HO, _WO)


# speedup vs baseline: 301.9555x; 1.0515x over previous
import functools

import jax
import jax.numpy as jnp
from jax import lax
from jax.experimental import pallas as pl
from jax.experimental.pallas import tpu as pltpu
from jax.experimental.pallas import tpu_sc as plsc

_B, _C, _H, _W = 8, 384, 32, 32
_NZ = 2
_HO = _H * (_NZ + 1) - _NZ
_WO = _W * (_NZ + 1) - _NZ
_PAIRS = _B * _C
_IN_PP = _H * _W
_OUT_PP = _HO * _WO

_NW = 32
_PAIRS_PER_W = _PAIRS // _NW
_UNIT_PAIRS = 4
_UNITS = _PAIRS_PER_W // _UNIT_PAIRS
_UNIT_IN = _UNIT_PAIRS * _IN_PP
_UNIT_OUT = _UNIT_PAIRS * _OUT_PP


@functools.partial(
    pl.kernel,
    out_type=jax.ShapeDtypeStruct((_PAIRS * _OUT_PP,), jnp.float32),
    mesh=plsc.VectorSubcoreMesh(core_axis_name="c", subcore_axis_name="s"),
    scratch_types=[
        pltpu.VMEM((2 * _UNIT_IN,), jnp.float32),
        pltpu.VMEM((2 * _UNIT_OUT,), jnp.float32),
        pltpu.SemaphoreType.DMA,
        pltpu.SemaphoreType.DMA,
        pltpu.SemaphoreType.DMA,
        pltpu.SemaphoreType.DMA,
    ],
    compiler_params=pltpu.CompilerParams(needs_layout_passes=False),
)
def _weave_sc(x_hbm, out_hbm, xbuf, obuf, sin0, sin1, sout0, sout1):
    nc = 2
    wid = lax.axis_index("s") * nc + lax.axis_index("c")
    base_pair = wid * _PAIRS_PER_W
    sin = (sin0, sin1)
    sout = (sout0, sout1)

    zeros16 = jnp.zeros((16,), jnp.float32)

    def _zero(i, c):
        obuf[pl.ds(i * 16, 16)] = zeros16
        return c

    lax.fori_loop(0, 2 * _UNIT_OUT // 16, _zero, 0)

    iota = lax.iota(jnp.int32, 16)
    col_lo = iota * 3
    col_hi = col_lo + 48

    def _in_start(u, p):
        start = (base_pair + u * _UNIT_PAIRS) * _IN_PP
        pltpu.async_copy(x_hbm.at[pl.ds(start, _UNIT_IN)],
                         xbuf.at[pl.ds(p * _UNIT_IN, _UNIT_IN)], sin[p])

    def _in_wait(p):
        pltpu.make_async_copy(
            x_hbm.at[pl.ds(0, _UNIT_IN)],
            xbuf.at[pl.ds(p * _UNIT_IN, _UNIT_IN)], sin[p]).wait()

    def _out_start(u, p):
        start = (base_pair + u * _UNIT_PAIRS) * _OUT_PP
        pltpu.async_copy(obuf.at[pl.ds(p * _UNIT_OUT, _UNIT_OUT)],
                         out_hbm.at[pl.ds(start, _UNIT_OUT)], sout[p])

    def _out_wait(p):
        pltpu.make_async_copy(
            obuf.at[pl.ds(p * _UNIT_OUT, _UNIT_OUT)],
            out_hbm.at[pl.ds(0, _UNIT_OUT)], sout[p]).wait()

    def _scatter(p):
        xoff = p * _UNIT_IN
        ooff = p * _UNIT_OUT
        for q in range(_UNIT_PAIRS):
            for r in range(_H):
                src = xoff + q * _IN_PP + r * _W
                row_lo = xbuf[pl.ds(src, 16)]
                row_hi = xbuf[pl.ds(src + 16, 16)]
                dst = ooff + q * _OUT_PP + (3 * r) * _WO
                plsc.store_scatter(obuf, [col_lo + dst], row_lo)
                plsc.store_scatter(obuf, [col_hi + dst], row_hi)

    _in_start(0, 0)
    _in_start(1, 1)
    for u in (0, 1):
        p = u
        _in_wait(p)
        _scatter(p)
        _out_start(u, p)
        _in_start(u + 2, p)

    def _steady(i, c):
        for p in (0, 1):
            u = 2 * i + p
            _out_wait(p)
            _in_wait(p)
            _scatter(p)
            _out_start(u, p)
            _in_start(u + 2, p)
        return c

    lax.fori_loop(1, (_UNITS - 2) // 2, _steady, 0)

    for u in (_UNITS - 2, _UNITS - 1):
        p = u % 2
        _out_wait(p)
        _in_wait(p)
        _scatter(p)
        _out_start(u, p)
    _out_wait(0)
    _out_wait(1)


def kernel(x):
    out_flat = _weave_sc(x.reshape(_PAIRS * _IN_PP))
    return out_flat.reshape(_B, _C, _HO, _WO)


# speedup vs baseline: 379.0684x; 1.2554x over previous
import functools

import jax
import jax.numpy as jnp
from jax import lax
from jax.experimental import pallas as pl
from jax.experimental.pallas import tpu as pltpu
from jax.experimental.pallas import tpu_sc as plsc

_B, _C, _H, _W = 8, 384, 32, 32
_NZ = 2
_HO = _H * (_NZ + 1) - _NZ
_WO = _W * (_NZ + 1) - _NZ

_NW = 32
_W_PER_B = _NW // _B
_C_PER_W = _C // _W_PER_B
_UNIT_C = 4
_UNITS = _C_PER_W // _UNIT_C


@functools.partial(
    pl.kernel,
    out_type=jax.ShapeDtypeStruct((_B, _C, _HO, _WO), jnp.float32),
    mesh=plsc.VectorSubcoreMesh(core_axis_name="c", subcore_axis_name="s"),
    scratch_types=[
        pltpu.VMEM((2, _UNIT_C, _H, _W), jnp.float32),
        pltpu.VMEM((2, _UNIT_C, _HO, _WO), jnp.float32),
        pltpu.SemaphoreType.DMA,
        pltpu.SemaphoreType.DMA,
        pltpu.SemaphoreType.DMA,
        pltpu.SemaphoreType.DMA,
    ],
    compiler_params=pltpu.CompilerParams(needs_layout_passes=False,
                                         use_tc_tiling_on_sc=False),
)
def _weave_sc(x_hbm, out_hbm, xbuf, obuf, sin0, sin1, sout0, sout1):
    nc = 2
    wid = lax.axis_index("s") * nc + lax.axis_index("c")
    b = wid // _W_PER_B
    c_base = (wid % _W_PER_B) * _C_PER_W
    sin = (sin0, sin1)
    sout = (sout0, sout1)

    iota = lax.iota(jnp.int32, 16)
    zeros16 = jnp.zeros((16,), jnp.float32)
    col_lo = iota * 3
    col_hi = col_lo + 48

    def _splat(v):
        return jnp.full((16,), v, jnp.int32)

    def _zero(r, c):
        row = _splat(r)
        for p in range(2):
            for q in range(_UNIT_C):
                for o in (0, 16, 32, 48, 64, _WO - 16):
                    plsc.store_scatter(
                        obuf, [_splat(p), _splat(q), row, iota + o], zeros16)
        return c

    lax.fori_loop(0, _HO, _zero, 0)

    def _in_start(u, p):
        pltpu.async_copy(x_hbm.at[b, pl.ds(c_base + u * _UNIT_C, _UNIT_C)],
                         xbuf.at[p], sin[p])

    def _in_wait(p):
        pltpu.make_async_copy(x_hbm.at[0, pl.ds(0, _UNIT_C)],
                              xbuf.at[p], sin[p]).wait()

    def _out_start(u, p):
        pltpu.async_copy(obuf.at[p],
                         out_hbm.at[b, pl.ds(c_base + u * _UNIT_C, _UNIT_C)],
                         sout[p])

    def _out_wait(p):
        pltpu.make_async_copy(obuf.at[p],
                              out_hbm.at[0, pl.ds(0, _UNIT_C)],
                              sout[p]).wait()

    def _scatter(p):
        sp = _splat(p)
        for q in range(_UNIT_C):
            sq = _splat(q)
            for r in range(_H):
                sr = _splat(r)
                row_lo = plsc.load_gather(xbuf, [sp, sq, sr, iota])
                row_hi = plsc.load_gather(xbuf, [sp, sq, sr, iota + 16])
                dr = _splat(3 * r)
                plsc.store_scatter(obuf, [sp, sq, dr, col_lo], row_lo)
                plsc.store_scatter(obuf, [sp, sq, dr, col_hi], row_hi)

    _in_start(0, 0)
    _in_start(1, 1)
    for u in (0, 1):
        p = u
        _in_wait(p)
        _scatter(p)
        _out_start(u, p)
        _in_start(u + 2, p)

    def _steady(i, c):
        for p in (0, 1):
            u = 2 * i + p
            _out_wait(p)
            _in_wait(p)
            _scatter(p)
            _out_start(u, p)
            _in_start(u + 2, p)
        return c

    lax.fori_loop(1, (_UNITS - 2) // 2, _steady, 0)

    for u in (_UNITS - 2, _UNITS - 1):
        p = u % 2
        _out_wait(p)
        _in_wait(p)
        _scatter(p)
        _out_start(u, p)
    _out_wait(0)
    _out_wait(1)


def kernel(x):
    return _weave_sc(x)


# speedup vs baseline: 626.3605x; 1.6524x over previous
import functools

import jax
import jax.numpy as jnp
from jax import lax
from jax.experimental import pallas as pl
from jax.experimental.pallas import tpu as pltpu
from jax.experimental.pallas import tpu_sc as plsc

_B, _C, _H, _W = 8, 384, 32, 32
_NZ = 2
_HO = _H * (_NZ + 1) - _NZ
_WO = _W * (_NZ + 1) - _NZ

_WP = 128
_HOP, _WOP = 96, 128
_WOB = 96

_NW = 32
_W_PER_B = _NW // _B
_C_PER_W = _C // _W_PER_B
_UNIT_C = 4
_UNITS = _C_PER_W // _UNIT_C


@functools.partial(
    pl.kernel,
    out_type=jax.ShapeDtypeStruct((_B, _C, _HOP, _WOP), jnp.float32),
    mesh=plsc.VectorSubcoreMesh(core_axis_name="c", subcore_axis_name="s"),
    scratch_types=[
        pltpu.VMEM((2, _UNIT_C, _H, _W), jnp.float32),
        pltpu.VMEM((2, _UNIT_C, _HO, _WOB), jnp.float32),
        pltpu.SemaphoreType.DMA,
        pltpu.SemaphoreType.DMA,
        pltpu.SemaphoreType.DMA,
        pltpu.SemaphoreType.DMA,
    ],
    compiler_params=pltpu.CompilerParams(needs_layout_passes=False,
                                         use_tc_tiling_on_sc=False),
)
def _weave_sc(x_hbm, out_hbm, xbuf, obuf, sin0, sin1, sout0, sout1):
    nc = 2
    wid = lax.axis_index("s") * nc + lax.axis_index("c")
    b = wid // _W_PER_B
    c_base = (wid % _W_PER_B) * _C_PER_W
    sin = (sin0, sin1)
    sout = (sout0, sout1)

    iota = lax.iota(jnp.int32, 16)
    zeros16 = jnp.zeros((16,), jnp.float32)
    col_lo = iota * 3
    col_hi = col_lo + 48

    def _splat(v):
        return jnp.full((16,), v, jnp.int32)

    def _zero(r, c):
        row = _splat(r)
        for p in range(2):
            for q in range(_UNIT_C):
                for o in (0, 16, 32, 48, 64, 80):
                    plsc.store_scatter(
                        obuf, [_splat(p), _splat(q), row, iota + o], zeros16)
        return c

    lax.fori_loop(0, _HO, _zero, 0)

    def _in_start(u, p):
        pltpu.async_copy(
            x_hbm.at[b, pl.ds(c_base + u * _UNIT_C, _UNIT_C),
                     pl.ds(0, _H), pl.ds(0, _W)],
            xbuf.at[p], sin[p])

    def _in_wait(p):
        pltpu.make_async_copy(
            x_hbm.at[0, pl.ds(0, _UNIT_C), pl.ds(0, _H), pl.ds(0, _W)],
            xbuf.at[p], sin[p]).wait()

    def _out_start(u, p):
        pltpu.async_copy(
            obuf.at[p],
            out_hbm.at[b, pl.ds(c_base + u * _UNIT_C, _UNIT_C),
                       pl.ds(0, _HO), pl.ds(0, _WOB)],
            sout[p])

    def _out_wait(p):
        pltpu.make_async_copy(
            obuf.at[p],
            out_hbm.at[0, pl.ds(0, _UNIT_C), pl.ds(0, _HO), pl.ds(0, _WOB)],
            sout[p]).wait()

    def _scatter(p):
        sp = _splat(p)
        for q in range(_UNIT_C):
            sq = _splat(q)
            for r in range(_H):
                sr = _splat(r)
                row_lo = plsc.load_gather(xbuf, [sp, sq, sr, iota])
                row_hi = plsc.load_gather(xbuf, [sp, sq, sr, iota + 16])
                dr = _splat(3 * r)
                plsc.store_scatter(obuf, [sp, sq, dr, col_lo], row_lo)
                plsc.store_scatter(obuf, [sp, sq, dr, col_hi], row_hi)

    _in_start(0, 0)
    _in_start(1, 1)
    for u in (0, 1):
        p = u
        _in_wait(p)
        _scatter(p)
        _out_start(u, p)
        _in_start(u + 2, p)

    def _steady(i, c):
        for p in (0, 1):
            u = 2 * i + p
            _out_wait(p)
            _in_wait(p)
            _scatter(p)
            _out_start(u, p)
            _in_start(u + 2, p)
        return c

    lax.fori_loop(1, (_UNITS - 2) // 2, _steady, 0)

    for u in (_UNITS - 2, _UNITS - 1):
        p = u % 2
        _out_wait(p)
        _in_wait(p)
        _scatter(p)
        _out_start(u, p)
    _out_wait(0)
    _out_wait(1)


def kernel(x):
    xp = jnp.pad(x, ((0, 0), (0, 0), (0, 0), (0, _WP - _W)))
    padded = _weave_sc(xp)
    return padded[:, :, :_HO, :_WO]
